# Initial kernel scaffold; baseline (speedup 1.0000x reference)
import jax
import jax.numpy as jnp
from jax.experimental import pallas as pl


def kernel(x, weight_matrix):
    raise NotImplementedError("write your pallas kernel here")


# speedup vs baseline: 4.0707x; 4.0707x over previous
import functools

import jax
import jax.numpy as jnp
from jax import lax
from jax.experimental import pallas as pl
from jax.experimental.pallas import tpu as pltpu
from jax.experimental.pallas import tpu_sc as plsc

BATCH = 4096
HIST = 50
EMBED_DIM = 64

NC = 2
NS = 16
NW = NC * NS

B_TOTAL = BATCH * HIST
B_PER_W = B_TOTAL // NW
G = 128
NCH = B_PER_W // G


def _gather_body(x_hbm, table_hbm, out_hbm, idx_v, rows_v, sem):
    cid = lax.axis_index("c")
    sid = lax.axis_index("s")
    wid = sid * NC + cid
    base = wid * B_PER_W

    pltpu.sync_copy(x_hbm.at[wid], idx_v)

    @pl.loop(0, NCH)
    def _(j):
        pltpu.async_copy(table_hbm.at[idx_v.at[j]], rows_v, sem).wait()
        pltpu.sync_copy(rows_v, out_hbm.at[pl.ds(base + j * G, G)])


@jax.jit
def _sc_gather(x_flat, table):
    mesh = plsc.VectorSubcoreMesh(core_axis_name="c", subcore_axis_name="s")
    f = functools.partial(
        pl.kernel,
        out_type=jax.ShapeDtypeStruct((B_TOTAL, EMBED_DIM), jnp.float32),
        mesh=mesh,
        scratch_types=[
            pltpu.VMEM((NCH, G), jnp.int32),
            pltpu.VMEM((G, EMBED_DIM), jnp.float32),
            pltpu.SemaphoreType.DMA,
        ],
        compiler_params=pltpu.CompilerParams(use_tc_tiling_on_sc=False),
    )(_gather_body)
    return f(x_flat.reshape(NW, NCH, G), table)


def _mask_body(x_ref, o_ref):
    o_ref[...] = x_ref[...] >= 1


@jax.jit
def _tc_mask(x):
    return pl.pallas_call(
        _mask_body,
        out_shape=jax.ShapeDtypeStruct((BATCH, HIST), jnp.bool_),
    )(x)


def kernel(x, weight_matrix):
    embeds = _sc_gather(x, weight_matrix).reshape(BATCH, HIST, EMBED_DIM)
    mask = _tc_mask(x)
    return embeds, mask


# speedup vs baseline: 4.2634x; 1.0473x over previous
import functools

import jax
import jax.numpy as jnp
from jax import lax
from jax.experimental import pallas as pl
from jax.experimental.pallas import tpu as pltpu
from jax.experimental.pallas import tpu_sc as plsc

BATCH = 4096
HIST = 50
EMBED_DIM = 64

NC = 2
NS = 16
NW = NC * NS

B_TOTAL = BATCH * HIST
B_PER_W = B_TOTAL // NW
G = 128
NCH = B_PER_W // G


def _gather_body(x_hbm, table_hbm, out_hbm, idx_v, rows0, rows1, sem0, sem1):
    cid = lax.axis_index("c")
    sid = lax.axis_index("s")
    wid = sid * NC + cid
    base = wid * B_PER_W

    pltpu.sync_copy(x_hbm.at[wid], idx_v)

    pltpu.async_copy(table_hbm.at[idx_v.at[0]], rows0, sem0)

    @pl.loop(0, NCH, step=2)
    def _(j):
        pltpu.make_async_copy(table_hbm.at[idx_v.at[j]], rows0, sem0).wait()
        pltpu.async_copy(table_hbm.at[idx_v.at[j + 1]], rows1, sem1)
        pltpu.sync_copy(rows0, out_hbm.at[pl.ds(base + j * G, G)])
        pltpu.make_async_copy(table_hbm.at[idx_v.at[j + 1]], rows1, sem1).wait()

        @pl.when(j + 2 < NCH)
        def _():
            pltpu.async_copy(table_hbm.at[idx_v.at[j + 2]], rows0, sem0)

        pltpu.sync_copy(rows1, out_hbm.at[pl.ds(base + (j + 1) * G, G)])


@jax.jit
def _sc_gather(x_flat, table):
    mesh = plsc.VectorSubcoreMesh(core_axis_name="c", subcore_axis_name="s")
    f = functools.partial(
        pl.kernel,
        out_type=jax.ShapeDtypeStruct((B_TOTAL, EMBED_DIM), jnp.float32),
        mesh=mesh,
        scratch_types=[
            pltpu.VMEM((NCH, G), jnp.int32),
            pltpu.VMEM((G, EMBED_DIM), jnp.float32),
            pltpu.VMEM((G, EMBED_DIM), jnp.float32),
            pltpu.SemaphoreType.DMA,
            pltpu.SemaphoreType.DMA,
        ],
        compiler_params=pltpu.CompilerParams(use_tc_tiling_on_sc=False),
    )(_gather_body)
    return f(x_flat.reshape(NW, NCH, G), table)


def _mask_body(x_ref, o_ref):
    o_ref[...] = x_ref[...] >= 1


@jax.jit
def _tc_mask(x):
    return pl.pallas_call(
        _mask_body,
        out_shape=jax.ShapeDtypeStruct((BATCH, HIST), jnp.bool_),
    )(x)


def kernel(x, weight_matrix):
    embeds = _sc_gather(x, weight_matrix).reshape(BATCH, HIST, EMBED_DIM)
    mask = _tc_mask(x)
    return embeds, mask
